# Initial kernel scaffold; baseline (speedup 1.0000x reference)
import functools

import jax
import jax.numpy as jnp
from jax import lax
from jax.experimental import pallas as pl
from jax.experimental.pallas import tpu as pltpu
from jax.experimental.pallas import tpu_sc as plsc

VOCAB = 100000
EMB = 64
SEQ = 200
BATCH = 1024
B = BATCH * SEQ

NC, NS, L = 2, 16, 16
NW = NC * NS
ROWS_PER_W = B // NW
CHUNK = 800
NCHUNK = ROWS_PER_W // CHUNK


def _comb_body(seg_ref, pos_ref, out_ref):
    seg = seg_ref[...]
    pos = pos_ref[...]
    out_ref[...] = (seg[:, None, :] + pos[None, :, :]).reshape(3 * SEQ, EMB)


def _make_comb(seg_table, pos_emb):
    return pl.pallas_call(
        _comb_body,
        out_shape=jax.ShapeDtypeStruct((3 * SEQ, EMB), jnp.float32),
    )(seg_table, pos_emb)


_mesh = plsc.VectorSubcoreMesh(core_axis_name="c", subcore_axis_name="s")


@functools.partial(
    pl.kernel,
    mesh=_mesh,
    out_type=jax.ShapeDtypeStruct((B, EMB), jnp.float32),
    scratch_types=[
        pltpu.VMEM((CHUNK,), jnp.int32),
        pltpu.VMEM((CHUNK,), jnp.int32),
        pltpu.VMEM((CHUNK,), jnp.int32),
        pltpu.VMEM((CHUNK, EMB), jnp.float32),
        pltpu.VMEM((CHUNK, EMB), jnp.float32),
        pltpu.SemaphoreType.DMA,
        pltpu.SemaphoreType.DMA,
    ],
)
def _sc_embed(inp_hbm, seg_hbm, tok_hbm, comb_hbm, out_hbm,
              idx_v, seg_v, cidx_v, tok_b, comb_b, sem0, sem1):
    wid = lax.axis_index("s") * NC + lax.axis_index("c")
    base = wid * ROWS_PER_W
    for c in range(NCHUNK):
        off = base + c * CHUNK
        pltpu.sync_copy(inp_hbm.at[pl.ds(off, CHUNK)], idx_v)
        pltpu.sync_copy(seg_hbm.at[pl.ds(off, CHUNK)], seg_v)

        def cidx_body(i, _):
            s = seg_v[pl.ds(i * L, L)]
            pos = (i * L + lax.iota(jnp.int32, L)) % SEQ
            cidx_v[pl.ds(i * L, L)] = s * SEQ + pos
            return 0

        lax.fori_loop(0, CHUNK // L, cidx_body, 0)

        cp0 = pltpu.async_copy(tok_hbm.at[idx_v], tok_b, sem0)
        cp1 = pltpu.async_copy(comb_hbm.at[cidx_v], comb_b, sem1)
        cp0.wait()
        cp1.wait()

        def add_body(r, _):
            for k in range(EMB // L):
                sl = pl.ds(k * L, L)
                tok_b[r, sl] = tok_b[r, sl] + comb_b[r, sl]
            return 0

        lax.fori_loop(0, CHUNK, add_body, 0)

        pltpu.sync_copy(tok_b, out_hbm.at[pl.ds(off, CHUNK)])


def kernel(input, segment_label, tok_table, seg_table, pos_emb):
    comb = _make_comb(seg_table, pos_emb)
    out = _sc_embed(input.reshape(-1), segment_label.reshape(-1),
                    tok_table, comb)
    return out.reshape(BATCH, SEQ, EMB)


# speedup vs baseline: 4.6101x; 4.6101x over previous
import functools

import jax
import jax.numpy as jnp
from jax import lax
from jax.experimental import pallas as pl
from jax.experimental.pallas import tpu as pltpu
from jax.experimental.pallas import tpu_sc as plsc

VOCAB = 100000
EMB = 64
SEQ = 200
BATCH = 1024
B = BATCH * SEQ

NC, NS, L = 2, 16, 16
NW = NC * NS
ROWS_PER_W = B // NW
CHUNK = 800
NCHUNK = ROWS_PER_W // CHUNK


def _comb_body(seg_ref, pos_ref, out_ref):
    seg = seg_ref[...]
    pos = pos_ref[...]
    out_ref[...] = (seg[:, None, :] + pos[None, :, :]).reshape(3 * SEQ, EMB)


def _make_comb(seg_table, pos_emb):
    return pl.pallas_call(
        _comb_body,
        out_shape=jax.ShapeDtypeStruct((3 * SEQ, EMB), jnp.float32),
    )(seg_table, pos_emb)


_mesh = plsc.VectorSubcoreMesh(core_axis_name="c", subcore_axis_name="s")


@functools.partial(
    pl.kernel,
    mesh=_mesh,
    compiler_params=pltpu.CompilerParams(use_tc_tiling_on_sc=False),
    out_type=jax.ShapeDtypeStruct((B, EMB), jnp.float32),
    scratch_types=[
        pltpu.VMEM((CHUNK,), jnp.int32),
        pltpu.VMEM((CHUNK,), jnp.int32),
        pltpu.VMEM((CHUNK,), jnp.int32),
        pltpu.VMEM((CHUNK, EMB), jnp.float32),
        pltpu.VMEM((CHUNK, EMB), jnp.float32),
        pltpu.SemaphoreType.DMA,
        pltpu.SemaphoreType.DMA,
    ],
)
def _sc_embed(inp_hbm, seg_hbm, tok_hbm, comb_hbm, out_hbm,
              idx_v, seg_v, cidx_v, tok_b, comb_b, sem0, sem1):
    wid = lax.axis_index("s") * NC + lax.axis_index("c")
    base = wid * ROWS_PER_W
    for c in range(NCHUNK):
        off = base + c * CHUNK
        pltpu.sync_copy(inp_hbm.at[pl.ds(off, CHUNK)], idx_v)
        pltpu.sync_copy(seg_hbm.at[pl.ds(off, CHUNK)], seg_v)

        def cidx_body(i, _):
            s = seg_v[pl.ds(i * L, L)]
            pos = (i * L + lax.iota(jnp.int32, L)) % SEQ
            cidx_v[pl.ds(i * L, L)] = s * SEQ + pos
            return 0

        lax.fori_loop(0, CHUNK // L, cidx_body, 0)

        cp0 = pltpu.async_copy(tok_hbm.at[idx_v], tok_b, sem0)
        cp1 = pltpu.async_copy(comb_hbm.at[cidx_v], comb_b, sem1)
        cp0.wait()
        cp1.wait()

        def add_body(r, _):
            for k in range(EMB // L):
                sl = pl.ds(k * L, L)
                tok_b[r, sl] = tok_b[r, sl] + comb_b[r, sl]
            return 0

        lax.fori_loop(0, CHUNK, add_body, 0)

        pltpu.sync_copy(tok_b, out_hbm.at[pl.ds(off, CHUNK)])


def kernel(input, segment_label, tok_table, seg_table, pos_emb):
    comb = _make_comb(seg_table, pos_emb)
    out = _sc_embed(input.reshape(-1), segment_label.reshape(-1),
                    tok_table, comb)
    return out.reshape(BATCH, SEQ, EMB)


# speedup vs baseline: 4.7279x; 1.0256x over previous
import functools

import jax
import jax.numpy as jnp
from jax import lax
from jax.experimental import pallas as pl
from jax.experimental.pallas import tpu as pltpu
from jax.experimental.pallas import tpu_sc as plsc

VOCAB = 100000
EMB = 64
SEQ = 200
BATCH = 1024
B = BATCH * SEQ

NC, NS, L = 2, 16, 16
NW = NC * NS
ROWS_PER_W = B // NW
CHUNK = 800
NCHUNK = ROWS_PER_W // CHUNK


def _comb_body(seg_ref, pos_ref, out_ref):
    seg = seg_ref[...]
    pos = pos_ref[...]
    out_ref[...] = (seg[:, None, :] + pos[None, :, :]).reshape(3 * SEQ, EMB)


def _make_comb(seg_table, pos_emb):
    return pl.pallas_call(
        _comb_body,
        out_shape=jax.ShapeDtypeStruct((3 * SEQ, EMB), jnp.float32),
    )(seg_table, pos_emb)


_mesh = plsc.VectorSubcoreMesh(core_axis_name="c", subcore_axis_name="s")


@functools.partial(
    pl.kernel,
    mesh=_mesh,
    compiler_params=pltpu.CompilerParams(use_tc_tiling_on_sc=False),
    out_type=jax.ShapeDtypeStruct((B, EMB), jnp.float32),
    scratch_types=[
        pltpu.VMEM((CHUNK,), jnp.int32),
        pltpu.VMEM((CHUNK,), jnp.int32),
        pltpu.VMEM((CHUNK,), jnp.int32),
        pltpu.VMEM((CHUNK, EMB), jnp.float32),
        pltpu.SemaphoreType.DMA,
        pltpu.SemaphoreType.DMA,
    ],
)
def _sc_embed(inp_hbm, seg_hbm, tok_hbm, comb_hbm, out_hbm,
              idx_v, seg_v, cidx_v, tok_b, sem0, sem1):
    wid = lax.axis_index("s") * NC + lax.axis_index("c")
    base = wid * ROWS_PER_W
    for c in range(NCHUNK):
        off = base + c * CHUNK
        pltpu.sync_copy(inp_hbm.at[pl.ds(off, CHUNK)], idx_v)
        pltpu.sync_copy(seg_hbm.at[pl.ds(off, CHUNK)], seg_v)

        def cidx_body(i, _):
            s = seg_v[pl.ds(i * L, L)]
            pos = (i * L + lax.iota(jnp.int32, L)) % SEQ
            cidx_v[pl.ds(i * L, L)] = s * SEQ + pos
            return 0

        lax.fori_loop(0, CHUNK // L, cidx_body, 0)

        pltpu.async_copy(tok_hbm.at[idx_v], tok_b, sem0).wait()
        pltpu.async_copy(comb_hbm.at[cidx_v], tok_b, sem1, add=True).wait()

        pltpu.sync_copy(tok_b, out_hbm.at[pl.ds(off, CHUNK)])


def kernel(input, segment_label, tok_table, seg_table, pos_emb):
    comb = _make_comb(seg_table, pos_emb)
    out = _sc_embed(input.reshape(-1), segment_label.reshape(-1),
                    tok_table, comb)
    return out.reshape(BATCH, SEQ, EMB)


# speedup vs baseline: 4.7775x; 1.0105x over previous
import functools

import jax
import jax.numpy as jnp
from jax import lax
from jax.experimental import pallas as pl
from jax.experimental.pallas import tpu as pltpu
from jax.experimental.pallas import tpu_sc as plsc

VOCAB = 100000
EMB = 64
SEQ = 200
BATCH = 1024
B = BATCH * SEQ

NC, NS, L = 2, 16, 16
NW = NC * NS
ROWS_PER_W = B // NW
CHUNK = 800
NCHUNK = ROWS_PER_W // CHUNK


def _comb_body(seg_ref, pos_ref, out_ref):
    seg = seg_ref[...]
    pos = pos_ref[...]
    out_ref[...] = (seg[:, None, :] + pos[None, :, :]).reshape(3 * SEQ, EMB)


def _make_comb(seg_table, pos_emb):
    return pl.pallas_call(
        _comb_body,
        out_shape=jax.ShapeDtypeStruct((3 * SEQ, EMB), jnp.float32),
    )(seg_table, pos_emb)


_mesh = plsc.VectorSubcoreMesh(core_axis_name="c", subcore_axis_name="s")


@functools.partial(
    pl.kernel,
    mesh=_mesh,
    compiler_params=pltpu.CompilerParams(use_tc_tiling_on_sc=False),
    out_type=jax.ShapeDtypeStruct((B, EMB), jnp.float32),
    scratch_types=[
        pltpu.VMEM((CHUNK,), jnp.int32),
        pltpu.VMEM((CHUNK,), jnp.int32),
        pltpu.VMEM((CHUNK,), jnp.int32),
        pltpu.VMEM((CHUNK,), jnp.int32),
        pltpu.VMEM((CHUNK,), jnp.int32),
        pltpu.VMEM((CHUNK,), jnp.int32),
        pltpu.VMEM((CHUNK, EMB), jnp.float32),
        pltpu.VMEM((CHUNK, EMB), jnp.float32),
        pltpu.SemaphoreType.DMA,
        pltpu.SemaphoreType.DMA,
        pltpu.SemaphoreType.DMA,
        pltpu.SemaphoreType.DMA,
        pltpu.SemaphoreType.DMA,
        pltpu.SemaphoreType.DMA,
    ],
)
def _sc_embed(inp_hbm, seg_hbm, tok_hbm, comb_hbm, out_hbm,
              idx0, idx1, seg0, seg1, cidx0, cidx1, tb0, tb1,
              semt0, semt1, sema0, sema1, semw0, semw1):
    idx = (idx0, idx1)
    seg = (seg0, seg1)
    cidx = (cidx0, cidx1)
    tb = (tb0, tb1)
    semt = (semt0, semt1)
    sema = (sema0, sema1)
    semw = (semw0, semw1)

    wid = lax.axis_index("s") * NC + lax.axis_index("c")
    base = wid * ROWS_PER_W

    def stage(c, p):
        off = base + c * CHUNK
        pltpu.sync_copy(inp_hbm.at[pl.ds(off, CHUNK)], idx[p])
        pltpu.sync_copy(seg_hbm.at[pl.ds(off, CHUNK)], seg[p])

        def cidx_body(i, _):
            s = seg[p][pl.ds(i * L, L)]
            pos = (i * L + lax.iota(jnp.int32, L)) % SEQ
            cidx[p][pl.ds(i * L, L)] = s * SEQ + pos
            return 0

        lax.fori_loop(0, CHUNK // L, cidx_body, 0)
        return pltpu.async_copy(tok_hbm.at[idx[p]], tb[p], semt[p])

    tok_cp = [None, None]
    wb_cp = [None, None]
    tok_cp[0] = stage(0, 0)
    for c in range(NCHUNK):
        p = c % 2
        q = p ^ 1
        tok_cp[p].wait()
        add_cp = pltpu.async_copy(comb_hbm.at[cidx[p]], tb[p], sema[p], add=True)
        if c + 1 < NCHUNK:
            if wb_cp[q] is not None:
                wb_cp[q].wait()
            tok_cp[q] = stage(c + 1, q)
        add_cp.wait()
        wb_cp[p] = pltpu.async_copy(tb[p], out_hbm.at[pl.ds(base + c * CHUNK, CHUNK)],
                                    semw[p])
    wb_cp[0].wait()
    wb_cp[1].wait()


def kernel(input, segment_label, tok_table, seg_table, pos_emb):
    comb = _make_comb(seg_table, pos_emb)
    out = _sc_embed(input.reshape(-1), segment_label.reshape(-1),
                    tok_table, comb)
    return out.reshape(BATCH, SEQ, EMB)
